# baseline (speedup 1.0000x reference)
import functools

import jax
import jax.numpy as jnp
from jax import lax
from jax.experimental import pallas as pl
from jax.experimental.pallas import tpu as pltpu
from jax.experimental.pallas import tpu_sc as plsc

_L = 16

_mesh = plsc.VectorSubcoreMesh(core_axis_name="c", subcore_axis_name="s")


@functools.partial(
    pl.kernel,
    mesh=_mesh,
    out_type=jax.ShapeDtypeStruct((_L,), jnp.float32),
    compiler_params=pltpu.CompilerParams(needs_layout_passes=False),
    scratch_types=[
        pltpu.VMEM((2 * _L,), jnp.int32),
        pltpu.VMEM((2 * _L,), jnp.int32),
        pltpu.VMEM((_L,), jnp.float32),
    ],
)
def _bit_compare_sc(a_hbm, b_hbm, out_hbm, a_v, b_v, r_v):
    cid = lax.axis_index("c")
    sid = lax.axis_index("s")

    @pl.when(jnp.logical_and(cid == 0, sid == 0))
    def _():
        pltpu.sync_copy(a_hbm, a_v)
        pltpu.sync_copy(b_hbm, b_v)
        a_hi = a_v[pl.ds(0, _L)]
        a_lo = a_v[pl.ds(_L, _L)]
        b_hi = b_v[pl.ds(0, _L)]
        b_lo = b_v[pl.ds(_L, _L)]
        neq_hi = a_hi != b_hi
        neq_lo = a_lo != b_lo
        use_hi = plsc.all_reduce_population_count(neq_hi) > 0
        idx = jnp.where(
            use_hi, plsc.all_reduce_ffs(neq_hi), plsc.all_reduce_ffs(neq_lo)
        )
        a_sel = jnp.where(use_hi, a_hi, a_lo)
        b_sel = jnp.where(use_hi, b_hi, b_lo)
        lane = lax.iota(jnp.int32, _L)
        hit = (lane == idx) & (a_sel == 0) & (b_sel == 1)
        r_v[...] = (plsc.all_reduce_population_count(hit) > 0).astype(
            jnp.float32
        )
        pltpu.sync_copy(r_v, out_hbm)


@jax.jit
def kernel(a, b):
    out = _bit_compare_sc(a, b)
    return out[0]


# speedup vs baseline: 1.1096x; 1.1096x over previous
import functools

import jax
import jax.numpy as jnp
from jax import lax
from jax.experimental import pallas as pl
from jax.experimental.pallas import tpu as pltpu
from jax.experimental.pallas import tpu_sc as plsc

_L = 16

_mesh = plsc.VectorSubcoreMesh(
    core_axis_name="c", subcore_axis_name="s", num_cores=1, num_subcores=1
)


@functools.partial(
    pl.kernel,
    mesh=_mesh,
    out_type=jax.ShapeDtypeStruct((_L,), jnp.float32),
    compiler_params=pltpu.CompilerParams(needs_layout_passes=False),
    scratch_types=[
        pltpu.VMEM((4 * _L,), jnp.int32),
        pltpu.VMEM((_L,), jnp.float32),
    ],
)
def _bit_compare_sc(ab_hbm, out_hbm, ab_v, r_v):
    pltpu.sync_copy(ab_hbm, ab_v)
    a_hi = ab_v[pl.ds(0, _L)]
    a_lo = ab_v[pl.ds(_L, _L)]
    b_hi = ab_v[pl.ds(2 * _L, _L)]
    b_lo = ab_v[pl.ds(3 * _L, _L)]
    neq_hi = a_hi != b_hi
    neq_lo = a_lo != b_lo
    use_hi = plsc.all_reduce_population_count(neq_hi) > 0
    idx = jnp.where(
        use_hi, plsc.all_reduce_ffs(neq_hi), plsc.all_reduce_ffs(neq_lo)
    )
    a_sel = jnp.where(use_hi, a_hi, a_lo)
    b_sel = jnp.where(use_hi, b_hi, b_lo)
    lane = lax.iota(jnp.int32, _L)
    hit = (lane == idx) & (a_sel == 0) & (b_sel == 1)
    r_v[...] = (plsc.all_reduce_population_count(hit) > 0).astype(jnp.float32)
    pltpu.sync_copy(r_v, out_hbm)


@jax.jit
def kernel(a, b):
    ab = jnp.concatenate([a, b])
    out = _bit_compare_sc(ab)
    return out[0]


# speedup vs baseline: 1.1934x; 1.0756x over previous
import functools

import jax
import jax.numpy as jnp
from jax.experimental import pallas as pl
from jax.experimental.pallas import tpu as pltpu
from jax.experimental.pallas import tpu_sc as plsc

_N = 32

_mesh = plsc.ScalarSubcoreMesh(axis_name="c", num_cores=1)


@functools.partial(
    pl.kernel,
    mesh=_mesh,
    out_type=jax.ShapeDtypeStruct((1,), jnp.float32),
    compiler_params=pltpu.CompilerParams(needs_layout_passes=False),
    scratch_types=[
        pltpu.SMEM((2 * _N,), jnp.int32),
        pltpu.SMEM((1,), jnp.float32),
    ],
)
def _bit_compare_scs(ab_hbm, out_hbm, ab_s, r_s):
    pltpu.sync_copy(ab_hbm, ab_s)
    less = jnp.int32(0)
    eq = jnp.int32(1)
    for i in range(_N):
        ai = ab_s[i]
        bi = ab_s[_N + i]
        less = less | (eq & (1 - ai) & bi)
        eq = eq & (1 - ((ai - bi) * (ai - bi)))
    r_s[0] = less.astype(jnp.float32)
    pltpu.sync_copy(r_s, out_hbm)


@jax.jit
def kernel(a, b):
    ab = jnp.concatenate([a, b])
    out = _bit_compare_scs(ab)
    return out[0]


# speedup vs baseline: 13.9778x; 11.7123x over previous
import jax
import jax.numpy as jnp
from jax.experimental import pallas as pl
from jax.experimental.pallas import tpu as pltpu

_N = 32


def _bit_compare_body(a_ref, b_ref, o_ref):
    less = jnp.int32(0)
    eq = jnp.int32(1)
    for i in range(_N):
        ai = a_ref[i]
        bi = b_ref[i]
        less = less | (eq & (1 - ai) & bi)
        eq = eq & (1 - ((ai - bi) * (ai - bi)))
    o_ref[0] = less.astype(jnp.float32)


@jax.jit
def kernel(a, b):
    out = pl.pallas_call(
        _bit_compare_body,
        out_shape=jax.ShapeDtypeStruct((1,), jnp.float32),
        in_specs=[
            pl.BlockSpec(memory_space=pltpu.SMEM),
            pl.BlockSpec(memory_space=pltpu.SMEM),
        ],
        out_specs=pl.BlockSpec(memory_space=pltpu.SMEM),
    )(a, b)
    return out[0]


# speedup vs baseline: 14.2949x; 1.0227x over previous
import jax
import jax.numpy as jnp
from jax.experimental import pallas as pl
from jax.experimental.pallas import tpu as pltpu

_N = 32


def _bit_compare_body(a_ref, b_ref, o_ref):
    a_hi = jnp.int32(0)
    b_hi = jnp.int32(0)
    a_lo = jnp.int32(0)
    b_lo = jnp.int32(0)
    for i in range(_N // 2):
        a_hi = a_hi + a_hi + a_ref[i]
        b_hi = b_hi + b_hi + b_ref[i]
        a_lo = a_lo + a_lo + a_ref[_N // 2 + i]
        b_lo = b_lo + b_lo + b_ref[_N // 2 + i]
    less = (a_hi < b_hi) | ((a_hi == b_hi) & (a_lo < b_lo))
    o_ref[0] = less.astype(jnp.float32)


@jax.jit
def kernel(a, b):
    out = pl.pallas_call(
        _bit_compare_body,
        out_shape=jax.ShapeDtypeStruct((1,), jnp.float32),
        in_specs=[
            pl.BlockSpec(memory_space=pltpu.SMEM),
            pl.BlockSpec(memory_space=pltpu.SMEM),
        ],
        out_specs=pl.BlockSpec(memory_space=pltpu.SMEM),
    )(a, b)
    return out[0]


# speedup vs baseline: 14.3392x; 1.0031x over previous
import jax
import jax.numpy as jnp
from jax.experimental import pallas as pl
from jax.experimental.pallas import tpu as pltpu

_N = 32


def _bit_compare_body(a_ref, b_ref, o_ref):
    a_hi = jnp.int32(0)
    b_hi = jnp.int32(0)
    a_lo = jnp.int32(0)
    b_lo = jnp.int32(0)
    for i in range(_N // 2):
        a_hi = a_hi + a_hi + a_ref[i]
        b_hi = b_hi + b_hi + b_ref[i]
        a_lo = a_lo + a_lo + a_ref[_N // 2 + i]
        b_lo = b_lo + b_lo + b_ref[_N // 2 + i]
    less = (a_hi < b_hi) | ((a_hi == b_hi) & (a_lo < b_lo))
    o_ref[...] = less.astype(jnp.float32)


@jax.jit
def kernel(a, b):
    return pl.pallas_call(
        _bit_compare_body,
        out_shape=jax.ShapeDtypeStruct((), jnp.float32),
        in_specs=[
            pl.BlockSpec(memory_space=pltpu.SMEM),
            pl.BlockSpec(memory_space=pltpu.SMEM),
        ],
        out_specs=pl.BlockSpec(memory_space=pltpu.SMEM),
    )(a, b)
